# Initial kernel scaffold; baseline (speedup 1.0000x reference)
import jax
import jax.numpy as jnp
from jax.experimental import pallas as pl


def kernel(xyz, points, W0, g0, b0, W1, g1, b1, W2, g2, b2):
    raise NotImplementedError("write your pallas kernel here")


# speedup vs baseline: 1.0049x; 1.0049x over previous
import jax
import jax.numpy as jnp
from jax import lax
from jax.experimental import pallas as pl

NPOINT = 512
RADIUS = 0.2
NSAMPLE = 32


def _fps(xyz, npoint):
    B, N, _ = xyz.shape

    def body(i, state):
        distance, farthest, centroids = state
        centroids = centroids.at[:, i].set(farthest)
        centroid = xyz[jnp.arange(B), farthest][:, None, :]
        dist = jnp.sum((xyz - centroid) ** 2, -1)
        distance = jnp.minimum(distance, dist)
        farthest = jnp.argmax(distance, -1).astype(jnp.int32)
        return (distance, farthest, centroids)

    distance = jnp.full((B, N), 1e10, dtype=xyz.dtype)
    farthest = jnp.zeros((B,), dtype=jnp.int32)
    centroids = jnp.zeros((B, npoint), dtype=jnp.int32)
    _, _, centroids = lax.fori_loop(0, npoint, body, (distance, farthest, centroids))
    return centroids


def _index_points(points, idx):
    B, N, C = points.shape
    offset = jnp.arange(B, dtype=idx.dtype).reshape((B,) + (1,) * (idx.ndim - 1)) * N
    flat = points.reshape(-1, C)
    out = flat[(idx + offset).reshape(-1)]
    return out.reshape(idx.shape + (C,))


def _square_distance(src, dst):
    dist = -2.0 * jnp.matmul(src, jnp.transpose(dst, (0, 2, 1)))
    dist += jnp.sum(src ** 2, -1)[:, :, None]
    dist += jnp.sum(dst ** 2, -1)[:, None, :]
    return dist


def _query_ball(radius, nsample, xyz, new_xyz):
    B, N, _ = xyz.shape
    S = new_xyz.shape[1]
    sqrdists = _square_distance(new_xyz, xyz)
    group_idx = jnp.broadcast_to(jnp.arange(N, dtype=jnp.int32), (B, S, N))
    group_idx = jnp.where(sqrdists > radius ** 2, N, group_idx)
    group_idx = jnp.sort(group_idx, axis=-1)[:, :, :nsample]
    first = group_idx[:, :, :1]
    first = jnp.where(first == N, 0, first)
    group_idx = jnp.where(group_idx == N, first, group_idx)
    return group_idx


def _bn(x, gamma, beta):
    mean = jnp.mean(x, axis=(0, 2, 3), keepdims=True)
    var = jnp.var(x, axis=(0, 2, 3), keepdims=True)
    xn = (x - mean) / jnp.sqrt(var + 1e-5)
    return xn * gamma.reshape(1, -1, 1, 1) + beta.reshape(1, -1, 1, 1)


def _pallas_id(x):
    def k(x_ref, o_ref):
        o_ref[...] = x_ref[...]
    return pl.pallas_call(k, out_shape=jax.ShapeDtypeStruct(x.shape, x.dtype))(x)


def kernel(xyz, points, W0, g0, b0, W1, g1, b1, W2, g2, b2):
    fps_idx = _fps(xyz, NPOINT)
    new_xyz = _index_points(xyz, fps_idx)
    idx = _query_ball(RADIUS, NSAMPLE, xyz, new_xyz)
    grouped_xyz = _index_points(xyz, idx) - new_xyz[:, :, None, :]
    grouped_pts = _index_points(jnp.transpose(points, (0, 2, 1)), idx)
    feats = jnp.concatenate([jnp.transpose(grouped_xyz, (0, 3, 1, 2)),
                             jnp.transpose(grouped_pts, (0, 3, 1, 2))], axis=1)
    for W, g, b in ((W0, g0, b0), (W1, g1, b1), (W2, g2, b2)):
        feats = jnp.einsum('oc,bsk->bosk'.replace('bsk', 'bcsk'), W, feats)
        feats = _bn(feats, g, b)
        feats = jax.nn.relu(feats)
    return _pallas_id(new_xyz), jnp.max(feats, axis=-1)


# speedup vs baseline: 2.4954x; 2.4832x over previous
import jax
import jax.numpy as jnp
from jax import lax
from jax.experimental import pallas as pl
from jax.experimental.pallas import tpu as pltpu

NPOINT = 512
RADIUS = 0.2
NSAMPLE = 32
B = 8
N = 4096


def _fps_body(xyzT_ref, cx_ref, cy_ref, cz_ref, dist_ref):
    x = xyzT_ref[0]
    y = xyzT_ref[1]
    z = xyzT_ref[2]
    lane = lax.broadcasted_iota(jnp.int32, (B, N), 1)

    def body(i, carry):
        far, dist = carry
        onehot = lane == far
        cx = jnp.sum(jnp.where(onehot, x, 0.0), axis=1, keepdims=True)
        cy = jnp.sum(jnp.where(onehot, y, 0.0), axis=1, keepdims=True)
        cz = jnp.sum(jnp.where(onehot, z, 0.0), axis=1, keepdims=True)
        cx_ref[pl.ds(i, 1), :] = cx.reshape(1, B)
        cy_ref[pl.ds(i, 1), :] = cy.reshape(1, B)
        cz_ref[pl.ds(i, 1), :] = cz.reshape(1, B)
        dx = x - cx
        dy = y - cy
        dz = z - cz
        d = (dx * dx + dz * dz) + dy * dy
        dist = jnp.minimum(dist, d)
        m = jnp.max(dist, axis=1, keepdims=True)
        far = jnp.min(jnp.where(dist == m, lane, N), axis=1, keepdims=True)
        return far, dist

    far0 = jnp.zeros((B, 1), jnp.int32)
    dist0 = jnp.full((B, N), 1e10, jnp.float32)
    lax.fori_loop(0, NPOINT, body, (far0, dist0))


def _fps(xyzT):
    out = jax.ShapeDtypeStruct((NPOINT, B), jnp.float32)
    return pl.pallas_call(
        _fps_body,
        out_shape=(out, out, out),
        scratch_shapes=[pltpu.VMEM((B, N), jnp.float32)],
    )(xyzT)


S = NPOINT
K = NSAMPLE
M = B * S * K
C1 = 64
C2 = 128
BLK = 4096
NBLK = M // BLK


def _q_body(pts_ref, xyzB_ref, w0_ref, q_ref):
    ptsb = pts_ref[0]
    xb = xyzB_ref[0]
    w0p = w0_ref[:, 3:67]
    w0x = w0_ref[:, 0:3]
    q = lax.dot_general(ptsb, w0p, (((0,), (1,)), ((), ())),
                        preferred_element_type=jnp.float32)
    qx = lax.dot_general(xb, w0x, (((0,), (1,)), ((), ())),
                         preferred_element_type=jnp.float32)
    q_ref[...] = q + qx


def _q_premul(points, xyzB, W0):
    blkN = 2048
    nj = N // blkN
    return pl.pallas_call(
        _q_body,
        grid=(B, nj),
        in_specs=[
            pl.BlockSpec((1, 64, blkN), lambda b, j: (b, 0, j)),
            pl.BlockSpec((1, 3, blkN), lambda b, j: (b, 0, j)),
            pl.BlockSpec((64, 67), lambda b, j: (0, 0)),
        ],
        out_specs=pl.BlockSpec((blkN, C1), lambda b, j: (b * nj + j, 0)),
        out_shape=jax.ShapeDtypeStruct((B * N, C1), jnp.float32),
    )(points, xyzB, W0)


def _c0_body(nx_ref, w0_ref, c0_ref):
    w0x = w0_ref[:, 0:3]
    c0_ref[...] = lax.dot_general(nx_ref[...], w0x, (((0,), (1,)), ((), ())),
                                  preferred_element_type=jnp.float32)


def _c0_premul(nxT, W0):
    return pl.pallas_call(
        _c0_body,
        out_shape=jax.ShapeDtypeStruct((B * S, C1), jnp.float32),
    )(nxT, W0)


def _expand_c0(c0blk):
    g = c0blk.shape[0]
    return jnp.broadcast_to(c0blk[:, None, :], (g, K, C1)).reshape(g * K, C1)


def _p1_body(g_ref, c0_ref, p_ref):
    y0 = g_ref[...] - _expand_c0(c0_ref[...])
    p_ref[0, 0, :] = jnp.sum(y0, axis=0)
    p_ref[0, 1, :] = jnp.sum(y0 * y0, axis=0)


def _bn_coefs(partials, g, b, nch):
    stats = jnp.sum(partials, axis=0)
    mean = stats[0:1, :] / M
    var = jnp.maximum(stats[1:2, :] / M - mean * mean, 0.0)
    scale = g / jnp.sqrt(var + 1e-5)
    shift = b - mean * scale
    return scale, shift


def _p2_body(g_ref, c0_ref, p0_ref, g0_ref, b0_ref, w1_ref, y1_ref, p_ref):
    scale, shift = _bn_coefs(p0_ref[...], g0_ref[...], b0_ref[...], C1)
    y0 = g_ref[...] - _expand_c0(c0_ref[...])
    x1 = jnp.maximum(y0 * scale + shift, 0.0)
    y1 = lax.dot_general(x1, w1_ref[...], (((1,), (1,)), ((), ())),
                         preferred_element_type=jnp.float32)
    y1_ref[...] = y1
    p_ref[0, 0, :] = jnp.sum(y1, axis=0)
    p_ref[0, 1, :] = jnp.sum(y1 * y1, axis=0)


def _p3_body(y1_ref, p1_ref, g1_ref, b1_ref, w2_ref, mx_ref, mn_ref, p_ref):
    scale, shift = _bn_coefs(p1_ref[...], g1_ref[...], b1_ref[...], C1)
    x2 = jnp.maximum(y1_ref[...] * scale + shift, 0.0)
    y2 = lax.dot_general(x2, w2_ref[...], (((1,), (1,)), ((), ())),
                         preferred_element_type=jnp.float32)
    y2g = y2.reshape(BLK // K, K, C2)
    mx_ref[...] = jnp.max(y2g, axis=1)
    mn_ref[...] = jnp.min(y2g, axis=1)
    p_ref[0, 0, :] = jnp.sum(y2, axis=0)
    p_ref[0, 1, :] = jnp.sum(y2 * y2, axis=0)


def _p4_body(mx_ref, mn_ref, p2_ref, g2_ref, b2_ref, o_ref):
    scale, shift = _bn_coefs(p2_ref[...], g2_ref[...], b2_ref[...], C2)
    y = jnp.where(scale >= 0.0, mx_ref[...], mn_ref[...])
    o_ref[...] = jnp.maximum(y * scale + shift, 0.0)


def _mlp(G, c0, g0, b0, W1, g1, b1, W2, g2, b2):
    gspec = pl.BlockSpec((BLK, C1), lambda i: (i, 0))
    c0spec = pl.BlockSpec((BLK // K, C1), lambda i: (i, 0))
    pspec1 = pl.BlockSpec((NBLK, 2, C1), lambda i: (0, 0, 0))
    pvec = lambda nch: pl.BlockSpec((1, 2, nch), lambda i: (i, 0, 0))
    full = lambda shp: pl.BlockSpec(shp, lambda i: tuple(0 for _ in shp))

    p0 = pl.pallas_call(
        _p1_body, grid=(NBLK,),
        in_specs=[gspec, c0spec],
        out_specs=pvec(C1),
        out_shape=jax.ShapeDtypeStruct((NBLK, 2, C1), jnp.float32),
    )(G, c0)

    y1, p1 = pl.pallas_call(
        _p2_body, grid=(NBLK,),
        in_specs=[gspec, c0spec, pspec1, full((1, C1)), full((1, C1)),
                  full((C1, C1))],
        out_specs=(gspec, pvec(C1)),
        out_shape=(jax.ShapeDtypeStruct((M, C1), jnp.float32),
                   jax.ShapeDtypeStruct((NBLK, 2, C1), jnp.float32)),
    )(G, c0, p0, g0.reshape(1, C1), b0.reshape(1, C1), W1)

    mx, mn, p2 = pl.pallas_call(
        _p3_body, grid=(NBLK,),
        in_specs=[gspec, pspec1, full((1, C1)), full((1, C1)), full((C2, C1))],
        out_specs=(pl.BlockSpec((BLK // K, C2), lambda i: (i, 0)),
                   pl.BlockSpec((BLK // K, C2), lambda i: (i, 0)),
                   pvec(C2)),
        out_shape=(jax.ShapeDtypeStruct((B * S, C2), jnp.float32),
                   jax.ShapeDtypeStruct((B * S, C2), jnp.float32),
                   jax.ShapeDtypeStruct((NBLK, 2, C2), jnp.float32)),
    )(y1, p1, g1.reshape(1, C1), b1.reshape(1, C1), W2)

    out = pl.pallas_call(
        _p4_body,
        out_shape=jax.ShapeDtypeStruct((B * S, C2), jnp.float32),
    )(mx, mn, p2, g2.reshape(1, C2), b2.reshape(1, C2))
    return out


def _square_distance(src, dst):
    dist = -2.0 * jnp.matmul(src, jnp.transpose(dst, (0, 2, 1)))
    dist += jnp.sum(src ** 2, -1)[:, :, None]
    dist += jnp.sum(dst ** 2, -1)[:, None, :]
    return dist


def _query_ball(radius, nsample, xyz, new_xyz):
    b, n, _ = xyz.shape
    s = new_xyz.shape[1]
    sqrdists = _square_distance(new_xyz, xyz)
    group_idx = jnp.broadcast_to(jnp.arange(n, dtype=jnp.int32), (b, s, n))
    group_idx = jnp.where(sqrdists > radius ** 2, n, group_idx)
    group_idx = jnp.sort(group_idx, axis=-1)[:, :, :nsample]
    first = group_idx[:, :, :1]
    first = jnp.where(first == n, 0, first)
    group_idx = jnp.where(group_idx == n, first, group_idx)
    return group_idx


def kernel(xyz, points, W0, g0, b0, W1, g1, b1, W2, g2, b2):
    xyzT = jnp.transpose(xyz, (2, 0, 1))
    cx, cy, cz = _fps(xyzT)
    new_xyz = jnp.stack([cx.T, cy.T, cz.T], axis=-1)

    idx = _query_ball(RADIUS, NSAMPLE, xyz, new_xyz)
    flat_idx = (idx + jnp.arange(B, dtype=jnp.int32)[:, None, None] * N).reshape(-1)

    q = _q_premul(points, jnp.transpose(xyz, (0, 2, 1)), W0)
    nxT = jnp.stack([cx.T.reshape(-1), cy.T.reshape(-1), cz.T.reshape(-1)])
    c0 = _c0_premul(nxT, W0)

    G = q[flat_idx]
    pooled = _mlp(G, c0, g0, b0, W1, g1, b1, W2, g2, b2)
    out = jnp.transpose(pooled.reshape(B, S, C2), (0, 2, 1))
    return new_xyz, out


# speedup vs baseline: 10.4657x; 4.1940x over previous
import dataclasses
import functools

import jax
import jax.numpy as jnp
from jax import lax
from jax.experimental import pallas as pl
from jax.experimental.pallas import tpu as pltpu
from jax.experimental.pallas import tpu_sc as plsc

NPOINT = 512
RADIUS = 0.2
NSAMPLE = 32
B = 8
N = 4096


def _fps_body(xyzT_ref, cx_ref, cy_ref, cz_ref, dist_ref):
    x = xyzT_ref[0]
    y = xyzT_ref[1]
    z = xyzT_ref[2]
    lane = lax.broadcasted_iota(jnp.int32, (B, N), 1)

    def body(i, carry):
        far, dist = carry
        onehot = lane == far
        cx = jnp.sum(jnp.where(onehot, x, 0.0), axis=1, keepdims=True)
        cy = jnp.sum(jnp.where(onehot, y, 0.0), axis=1, keepdims=True)
        cz = jnp.sum(jnp.where(onehot, z, 0.0), axis=1, keepdims=True)
        cx_ref[pl.ds(i, 1), :] = cx.reshape(1, B)
        cy_ref[pl.ds(i, 1), :] = cy.reshape(1, B)
        cz_ref[pl.ds(i, 1), :] = cz.reshape(1, B)
        dx = x - cx
        dy = y - cy
        dz = z - cz
        d = (dx * dx + dz * dz) + dy * dy
        dist = jnp.minimum(dist, d)
        m = jnp.max(dist, axis=1, keepdims=True)
        far = jnp.min(jnp.where(dist == m, lane, N), axis=1, keepdims=True)
        return far, dist

    far0 = jnp.zeros((B, 1), jnp.int32)
    dist0 = jnp.full((B, N), 1e10, jnp.float32)
    lax.fori_loop(0, NPOINT, body, (far0, dist0))


def _fps(xyzT):
    out = jax.ShapeDtypeStruct((NPOINT, B), jnp.float32)
    return pl.pallas_call(
        _fps_body,
        out_shape=(out, out, out),
        scratch_shapes=[pltpu.VMEM((B, N), jnp.float32)],
    )(xyzT)


S = NPOINT
K = NSAMPLE
M = B * S * K
C1 = 64
C2 = 128
BLK = 4096
NBLK = M // BLK


def _q_body(pts_ref, xyzB_ref, w0_ref, q_ref):
    ptsb = pts_ref[0]
    xb = xyzB_ref[0]
    w0p = w0_ref[:, 3:67]
    w0x = w0_ref[:, 0:3]
    q = lax.dot_general(ptsb, w0p, (((0,), (1,)), ((), ())),
                        preferred_element_type=jnp.float32)
    qx = lax.dot_general(xb, w0x, (((0,), (1,)), ((), ())),
                         preferred_element_type=jnp.float32)
    q_ref[...] = q + qx


def _q_premul(points, xyzB, W0):
    blkN = 2048
    nj = N // blkN
    return pl.pallas_call(
        _q_body,
        grid=(B, nj),
        in_specs=[
            pl.BlockSpec((1, 64, blkN), lambda b, j: (b, 0, j)),
            pl.BlockSpec((1, 3, blkN), lambda b, j: (b, 0, j)),
            pl.BlockSpec((64, 67), lambda b, j: (0, 0)),
        ],
        out_specs=pl.BlockSpec((blkN, C1), lambda b, j: (b * nj + j, 0)),
        out_shape=jax.ShapeDtypeStruct((B * N, C1), jnp.float32),
    )(points, xyzB, W0)


def _c0_body(nx_ref, w0_ref, c0_ref):
    w0x = w0_ref[:, 0:3]
    c0_ref[...] = lax.dot_general(nx_ref[...], w0x, (((0,), (1,)), ((), ())),
                                  preferred_element_type=jnp.float32)


def _c0_premul(nxT, W0):
    return pl.pallas_call(
        _c0_body,
        out_shape=jax.ShapeDtypeStruct((B * S, C1), jnp.float32),
    )(nxT, W0)


def _expand_c0(c0blk):
    g = c0blk.shape[0]
    return jnp.broadcast_to(c0blk[:, None, :], (g, K, C1)).reshape(g * K, C1)


def _p1_body(g_ref, c0_ref, p_ref):
    y0 = g_ref[...] - _expand_c0(c0_ref[...])
    p_ref[0, 0, :] = jnp.sum(y0, axis=0)
    p_ref[0, 1, :] = jnp.sum(y0 * y0, axis=0)


def _bn_coefs(partials, g, b, nch):
    stats = jnp.sum(partials, axis=0)
    mean = stats[0:1, :] / M
    var = jnp.maximum(stats[1:2, :] / M - mean * mean, 0.0)
    scale = g / jnp.sqrt(var + 1e-5)
    shift = b - mean * scale
    return scale, shift


def _p2_body(g_ref, c0_ref, p0_ref, g0_ref, b0_ref, w1_ref, y1_ref, p_ref):
    scale, shift = _bn_coefs(p0_ref[...], g0_ref[...], b0_ref[...], C1)
    y0 = g_ref[...] - _expand_c0(c0_ref[...])
    x1 = jnp.maximum(y0 * scale + shift, 0.0)
    y1 = lax.dot_general(x1, w1_ref[...], (((1,), (1,)), ((), ())),
                         preferred_element_type=jnp.float32)
    y1_ref[...] = y1
    p_ref[0, 0, :] = jnp.sum(y1, axis=0)
    p_ref[0, 1, :] = jnp.sum(y1 * y1, axis=0)


def _p3_body(y1_ref, p1_ref, g1_ref, b1_ref, w2_ref, mx_ref, mn_ref, p_ref):
    scale, shift = _bn_coefs(p1_ref[...], g1_ref[...], b1_ref[...], C1)
    x2 = jnp.maximum(y1_ref[...] * scale + shift, 0.0)
    y2 = lax.dot_general(x2, w2_ref[...], (((1,), (1,)), ((), ())),
                         preferred_element_type=jnp.float32)
    y2g = y2.reshape(BLK // K, K, C2)
    mx_ref[...] = jnp.max(y2g, axis=1)
    mn_ref[...] = jnp.min(y2g, axis=1)
    p_ref[0, 0, :] = jnp.sum(y2, axis=0)
    p_ref[0, 1, :] = jnp.sum(y2 * y2, axis=0)


def _p4_body(mx_ref, mn_ref, p2_ref, g2_ref, b2_ref, o_ref):
    scale, shift = _bn_coefs(p2_ref[...], g2_ref[...], b2_ref[...], C2)
    y = jnp.where(scale >= 0.0, mx_ref[...], mn_ref[...])
    o_ref[...] = jnp.maximum(y * scale + shift, 0.0)


def _mlp(G, c0, g0, b0, W1, g1, b1, W2, g2, b2):
    gspec = pl.BlockSpec((BLK, C1), lambda i: (i, 0))
    c0spec = pl.BlockSpec((BLK // K, C1), lambda i: (i, 0))
    pspec1 = pl.BlockSpec((NBLK, 2, C1), lambda i: (0, 0, 0))
    pvec = lambda nch: pl.BlockSpec((1, 2, nch), lambda i: (i, 0, 0))
    full = lambda shp: pl.BlockSpec(shp, lambda i: tuple(0 for _ in shp))

    p0 = pl.pallas_call(
        _p1_body, grid=(NBLK,),
        in_specs=[gspec, c0spec],
        out_specs=pvec(C1),
        out_shape=jax.ShapeDtypeStruct((NBLK, 2, C1), jnp.float32),
    )(G, c0)

    y1, p1 = pl.pallas_call(
        _p2_body, grid=(NBLK,),
        in_specs=[gspec, c0spec, pspec1, full((1, C1)), full((1, C1)),
                  full((C1, C1))],
        out_specs=(gspec, pvec(C1)),
        out_shape=(jax.ShapeDtypeStruct((M, C1), jnp.float32),
                   jax.ShapeDtypeStruct((NBLK, 2, C1), jnp.float32)),
    )(G, c0, p0, g0.reshape(1, C1), b0.reshape(1, C1), W1)

    mx, mn, p2 = pl.pallas_call(
        _p3_body, grid=(NBLK,),
        in_specs=[gspec, pspec1, full((1, C1)), full((1, C1)), full((C2, C1))],
        out_specs=(pl.BlockSpec((BLK // K, C2), lambda i: (i, 0)),
                   pl.BlockSpec((BLK // K, C2), lambda i: (i, 0)),
                   pvec(C2)),
        out_shape=(jax.ShapeDtypeStruct((B * S, C2), jnp.float32),
                   jax.ShapeDtypeStruct((B * S, C2), jnp.float32),
                   jax.ShapeDtypeStruct((NBLK, 2, C2), jnp.float32)),
    )(y1, p1, g1.reshape(1, C1), b1.reshape(1, C1), W2)

    out = pl.pallas_call(
        _p4_body,
        out_shape=jax.ShapeDtypeStruct((B * S, C2), jnp.float32),
    )(mx, mn, p2, g2.reshape(1, C2), b2.reshape(1, C2))
    return out


def _sc_params(tc_tiling=True):
    cp = pltpu.CompilerParams()
    if "needs_layout_passes" in pltpu.CompilerParams.__dataclass_fields__:
        cp = dataclasses.replace(cp, needs_layout_passes=False)
    if not tc_tiling:
        cp = dataclasses.replace(cp, use_tc_tiling_on_sc=False)
    return cp


NWORK = 32
CPW = (B * S) // NWORK
NCHUNK = N // 16


SB = 256


def _dist_body(nx_ref, xt_ref, m_ref):
    src = nx_ref[0]
    dst = xt_ref[0]
    mm = lax.dot_general(src, dst, (((1,), (0,)), ((), ())),
                         preferred_element_type=jnp.float32)
    s0 = src[:, 0] * src[:, 0]
    s1 = src[:, 1] * src[:, 1]
    s2 = src[:, 2] * src[:, 2]
    a2 = ((s0 + s1) + s2).reshape(SB, 1)
    d0 = dst[0] * dst[0]
    d1 = dst[1] * dst[1]
    d2 = dst[2] * dst[2]
    b2 = ((d0 + d1) + d2).reshape(1, N)
    d = (-2.0 * mm + a2) + b2
    m_ref[...] = jnp.logical_not(d > RADIUS ** 2).astype(jnp.int32)


def _dist_mask(new_xyz, xyzB):
    nj = S // SB
    return pl.pallas_call(
        _dist_body,
        grid=(B, nj),
        in_specs=[
            pl.BlockSpec((1, SB, 3), lambda b, j: (b, j, 0)),
            pl.BlockSpec((1, 3, N), lambda b, j: (b, 0, 0)),
        ],
        out_specs=pl.BlockSpec((SB, N), lambda b, j: (b * nj + j, 0)),
        out_shape=jax.ShapeDtypeStruct((B * S, N), jnp.int32),
    )(new_xyz, xyzB)


ROWB = 8
NCH = CPW // ROWB


def _sc_ball(maskf):
    mesh = plsc.VectorSubcoreMesh(core_axis_name="c", subcore_axis_name="s")

    @functools.partial(
        pl.kernel, mesh=mesh, compiler_params=_sc_params(),
        out_type=jax.ShapeDtypeStruct((M,), jnp.int32),
        scratch_types=[
            pltpu.VMEM((ROWB * N,), jnp.int32),
            pltpu.VMEM((ROWB * N,), jnp.int32),
            pltpu.VMEM((64,), jnp.int32),
            pltpu.VMEM((CPW * K,), jnp.int32),
            pltpu.SemaphoreType.DMA,
            pltpu.SemaphoreType.DMA,
        ],
    )
    def k(maskf_hbm, idx_hbm, mb0, mb1, buf, obuf, s0, s1):
        wid = lax.axis_index("s") * 2 + lax.axis_index("c")
        b = wid // (NWORK // B)
        cb = wid * CPW
        iota = lax.iota(jnp.int32, 16)
        bN = b * N

        def chunk_src(ci):
            return maskf_hbm.at[pl.ds((cb + ci * ROWB) * N, ROWB * N)]

        def do_rows(mbuf, ci):
            @pl.loop(0, ROWB)
            def _(r):
                rbase = r * N

                def cond(st):
                    c, cnt = st
                    return (cnt < K) & (c < NCHUNK)

                def body(st):
                    c, cnt = st
                    sel = mbuf[pl.ds(rbase + c * 16, 16)]
                    ps = plsc.cumsum(sel)
                    plsc.store_scatter(buf, [(cnt - 1) + ps], iota + c * 16,
                                       mask=sel > 0)
                    return c + 1, cnt + jnp.max(ps)

                _, cnt = lax.while_loop(cond, body,
                                        (jnp.int32(0), jnp.int32(0)))
                firstv = plsc.load_gather(buf, [jnp.zeros((16,), jnp.int32)])
                j = ci * ROWB + r
                for g in range(K // 16):
                    lanes = iota + (g * 16)
                    v = buf[pl.ds(g * 16, 16)]
                    outg = jnp.where(lanes < cnt, v, firstv) + bN
                    plsc.store_scatter(obuf, [lanes + (j * K)], outg)

        pltpu.async_copy(chunk_src(0), mb0, s0)

        @pl.loop(0, NCH, step=2)
        def _(ci):
            pltpu.async_copy(chunk_src(ci + 1), mb1, s1)
            pltpu.make_async_copy(chunk_src(ci), mb0, s0).wait()
            do_rows(mb0, ci)

            @pl.when(ci + 2 < NCH)
            def _():
                pltpu.async_copy(chunk_src(ci + 2), mb0, s0)

            pltpu.make_async_copy(chunk_src(ci + 1), mb1, s1).wait()
            do_rows(mb1, ci + 1)

        pltpu.sync_copy(obuf, idx_hbm.at[pl.ds(wid * CPW * K, CPW * K)])

    return k(maskf)


def _sc_gather(q, idx):
    mesh = plsc.VectorSubcoreMesh(core_axis_name="c", subcore_axis_name="s")
    RPW = M // NWORK
    CH = 128

    @functools.partial(
        pl.kernel, mesh=mesh, compiler_params=_sc_params(tc_tiling=False),
        out_type=jax.ShapeDtypeStruct((M, C1), jnp.float32),
        scratch_types=[
            pltpu.VMEM((RPW,), jnp.int32),
            pltpu.VMEM((CH, C1), jnp.float32),
            pltpu.VMEM((CH, C1), jnp.float32),
            pltpu.SemaphoreType.DMA,
            pltpu.SemaphoreType.DMA,
        ],
    )
    def k(q_hbm, idx_hbm, g_hbm, idxv, rb0, rb1, s0, s1):
        wid = lax.axis_index("s") * 2 + lax.axis_index("c")
        base = wid * RPW
        pltpu.sync_copy(idx_hbm.at[pl.ds(base, RPW)], idxv)

        @pl.loop(0, RPW // CH, step=2)
        def _(c):
            h0 = pltpu.async_copy(q_hbm.at[idxv.at[pl.ds(c * CH, CH)]], rb0, s0)
            h1 = pltpu.async_copy(
                q_hbm.at[idxv.at[pl.ds((c + 1) * CH, CH)]], rb1, s1)
            h0.wait()
            pltpu.sync_copy(rb0, g_hbm.at[pl.ds(base + c * CH, CH)])
            h1.wait()
            pltpu.sync_copy(rb1, g_hbm.at[pl.ds(base + (c + 1) * CH, CH)])

    return k(q, idx)


def kernel(xyz, points, W0, g0, b0, W1, g1, b1, W2, g2, b2):
    xyzT = jnp.transpose(xyz, (2, 0, 1))
    cx, cy, cz = _fps(xyzT)
    new_xyz = jnp.stack([cx.T, cy.T, cz.T], axis=-1)

    xyzB = jnp.transpose(xyz, (0, 2, 1))
    q = _q_premul(points, xyzB, W0)
    nxT = jnp.stack([cx.T.reshape(-1), cy.T.reshape(-1), cz.T.reshape(-1)])
    c0 = _c0_premul(nxT, W0)

    mask = _dist_mask(new_xyz, xyzB)
    flat_idx = _sc_ball(mask.reshape(-1))
    G = _sc_gather(q, flat_idx)
    pooled = _mlp(G, c0, g0, b0, W1, g1, b1, W2, g2, b2)
    out = jnp.transpose(pooled.reshape(B, S, C2), (0, 2, 1))
    return new_xyz, out


# speedup vs baseline: 13.4200x; 1.2823x over previous
import dataclasses
import functools

import jax
import jax.numpy as jnp
from jax import lax
from jax.experimental import pallas as pl
from jax.experimental.pallas import tpu as pltpu
from jax.experimental.pallas import tpu_sc as plsc

NPOINT = 512
RADIUS = 0.2
NSAMPLE = 32
B = 8
N = 4096


def _fps_body(xyzT_ref, cx_ref, cy_ref, cz_ref, dist_ref):
    x = xyzT_ref[0]
    y = xyzT_ref[1]
    z = xyzT_ref[2]
    lane = lax.broadcasted_iota(jnp.int32, (B, N), 1)

    def body(i, carry):
        far, dist = carry
        onehot = lane == far
        cx = jnp.sum(jnp.where(onehot, x, 0.0), axis=1, keepdims=True)
        cy = jnp.sum(jnp.where(onehot, y, 0.0), axis=1, keepdims=True)
        cz = jnp.sum(jnp.where(onehot, z, 0.0), axis=1, keepdims=True)
        cx_ref[pl.ds(i, 1), :] = cx.reshape(1, B)
        cy_ref[pl.ds(i, 1), :] = cy.reshape(1, B)
        cz_ref[pl.ds(i, 1), :] = cz.reshape(1, B)
        dx = x - cx
        dy = y - cy
        dz = z - cz
        d = (dx * dx + dz * dz) + dy * dy
        dist = jnp.minimum(dist, d)
        m = jnp.max(dist, axis=1, keepdims=True)
        far = jnp.min(jnp.where(dist == m, lane, N), axis=1, keepdims=True)
        return far, dist

    far0 = jnp.zeros((B, 1), jnp.int32)
    dist0 = jnp.full((B, N), 1e10, jnp.float32)
    lax.fori_loop(0, NPOINT, body, (far0, dist0))


def _fps(xyzT):
    out = jax.ShapeDtypeStruct((NPOINT, B), jnp.float32)
    return pl.pallas_call(
        _fps_body,
        out_shape=(out, out, out),
        scratch_shapes=[pltpu.VMEM((B, N), jnp.float32)],
    )(xyzT)


S = NPOINT
K = NSAMPLE
M = B * S * K
C1 = 64
C2 = 128
BLK = 4096
NBLK = M // BLK


def _q_body(pts_ref, xyzB_ref, w0_ref, q_ref):
    ptsb = pts_ref[0]
    xb = xyzB_ref[0]
    w0p = w0_ref[:, 3:67]
    w0x = w0_ref[:, 0:3]
    q = lax.dot_general(ptsb, w0p, (((0,), (1,)), ((), ())),
                        preferred_element_type=jnp.float32)
    qx = lax.dot_general(xb, w0x, (((0,), (1,)), ((), ())),
                         preferred_element_type=jnp.float32)
    q_ref[...] = q + qx


def _q_premul(points, xyzB, W0):
    blkN = 2048
    nj = N // blkN
    return pl.pallas_call(
        _q_body,
        grid=(B, nj),
        in_specs=[
            pl.BlockSpec((1, 64, blkN), lambda b, j: (b, 0, j)),
            pl.BlockSpec((1, 3, blkN), lambda b, j: (b, 0, j)),
            pl.BlockSpec((64, 67), lambda b, j: (0, 0)),
        ],
        out_specs=pl.BlockSpec((blkN, C1), lambda b, j: (b * nj + j, 0)),
        out_shape=jax.ShapeDtypeStruct((B * N, C1), jnp.float32),
    )(points, xyzB, W0)


def _c0_body(nx_ref, w0_ref, c0_ref):
    w0x = w0_ref[:, 0:3]
    c0_ref[...] = lax.dot_general(nx_ref[...], w0x, (((0,), (1,)), ((), ())),
                                  preferred_element_type=jnp.float32)


def _c0_premul(nxT, W0):
    return pl.pallas_call(
        _c0_body,
        out_shape=jax.ShapeDtypeStruct((B * S, C1), jnp.float32),
    )(nxT, W0)


def _expand_c0(c0blk):
    g = c0blk.shape[0]
    return jnp.broadcast_to(c0blk[:, None, :], (g, K, C1)).reshape(g * K, C1)


def _p1_body(g_ref, c0_ref, p_ref):
    y0 = g_ref[...] - _expand_c0(c0_ref[...])
    p_ref[0, 0, :] = jnp.sum(y0, axis=0)
    p_ref[0, 1, :] = jnp.sum(y0 * y0, axis=0)


def _bn_coefs(partials, g, b, nch):
    stats = jnp.sum(partials, axis=0)
    mean = stats[0:1, :] / M
    var = jnp.maximum(stats[1:2, :] / M - mean * mean, 0.0)
    scale = g / jnp.sqrt(var + 1e-5)
    shift = b - mean * scale
    return scale, shift


def _p2_body(g_ref, c0_ref, p0_ref, g0_ref, b0_ref, w1_ref, y1_ref, p_ref):
    scale, shift = _bn_coefs(p0_ref[...], g0_ref[...], b0_ref[...], C1)
    y0 = g_ref[...] - _expand_c0(c0_ref[...])
    x1 = jnp.maximum(y0 * scale + shift, 0.0)
    y1 = lax.dot_general(x1, w1_ref[...], (((1,), (1,)), ((), ())),
                         preferred_element_type=jnp.float32)
    y1_ref[...] = y1
    p_ref[0, 0, :] = jnp.sum(y1, axis=0)
    p_ref[0, 1, :] = jnp.sum(y1 * y1, axis=0)


def _p3_body(y1_ref, p1_ref, g1_ref, b1_ref, w2_ref, mx_ref, mn_ref, p_ref):
    scale, shift = _bn_coefs(p1_ref[...], g1_ref[...], b1_ref[...], C1)
    x2 = jnp.maximum(y1_ref[...] * scale + shift, 0.0)
    y2 = lax.dot_general(x2, w2_ref[...], (((1,), (1,)), ((), ())),
                         preferred_element_type=jnp.float32)
    y2g = y2.reshape(BLK // K, K, C2)
    mx_ref[...] = jnp.max(y2g, axis=1)
    mn_ref[...] = jnp.min(y2g, axis=1)
    p_ref[0, 0, :] = jnp.sum(y2, axis=0)
    p_ref[0, 1, :] = jnp.sum(y2 * y2, axis=0)


def _p4_body(mx_ref, mn_ref, p2_ref, g2_ref, b2_ref, o_ref):
    scale, shift = _bn_coefs(p2_ref[...], g2_ref[...], b2_ref[...], C2)
    y = jnp.where(scale >= 0.0, mx_ref[...], mn_ref[...])
    o_ref[...] = jnp.maximum(y * scale + shift, 0.0)


def _mlp(G, c0, g0, b0, W1, g1, b1, W2, g2, b2):
    gspec = pl.BlockSpec((BLK, C1), lambda i: (i, 0))
    c0spec = pl.BlockSpec((BLK // K, C1), lambda i: (i, 0))
    pspec1 = pl.BlockSpec((NBLK, 2, C1), lambda i: (0, 0, 0))
    pvec = lambda nch: pl.BlockSpec((1, 2, nch), lambda i: (i, 0, 0))
    full = lambda shp: pl.BlockSpec(shp, lambda i: tuple(0 for _ in shp))

    p0 = pl.pallas_call(
        _p1_body, grid=(NBLK,),
        in_specs=[gspec, c0spec],
        out_specs=pvec(C1),
        out_shape=jax.ShapeDtypeStruct((NBLK, 2, C1), jnp.float32),
    )(G, c0)

    y1, p1 = pl.pallas_call(
        _p2_body, grid=(NBLK,),
        in_specs=[gspec, c0spec, pspec1, full((1, C1)), full((1, C1)),
                  full((C1, C1))],
        out_specs=(gspec, pvec(C1)),
        out_shape=(jax.ShapeDtypeStruct((M, C1), jnp.float32),
                   jax.ShapeDtypeStruct((NBLK, 2, C1), jnp.float32)),
    )(G, c0, p0, g0.reshape(1, C1), b0.reshape(1, C1), W1)

    mx, mn, p2 = pl.pallas_call(
        _p3_body, grid=(NBLK,),
        in_specs=[gspec, pspec1, full((1, C1)), full((1, C1)), full((C2, C1))],
        out_specs=(pl.BlockSpec((BLK // K, C2), lambda i: (i, 0)),
                   pl.BlockSpec((BLK // K, C2), lambda i: (i, 0)),
                   pvec(C2)),
        out_shape=(jax.ShapeDtypeStruct((B * S, C2), jnp.float32),
                   jax.ShapeDtypeStruct((B * S, C2), jnp.float32),
                   jax.ShapeDtypeStruct((NBLK, 2, C2), jnp.float32)),
    )(y1, p1, g1.reshape(1, C1), b1.reshape(1, C1), W2)

    out = pl.pallas_call(
        _p4_body,
        out_shape=jax.ShapeDtypeStruct((B * S, C2), jnp.float32),
    )(mx, mn, p2, g2.reshape(1, C2), b2.reshape(1, C2))
    return out


def _sc_params(tc_tiling=True):
    cp = pltpu.CompilerParams()
    if "needs_layout_passes" in pltpu.CompilerParams.__dataclass_fields__:
        cp = dataclasses.replace(cp, needs_layout_passes=False)
    if not tc_tiling:
        cp = dataclasses.replace(cp, use_tc_tiling_on_sc=False)
    return cp


NWORK = 32
CPW = (B * S) // NWORK
NCHUNK = N // 16


SB = 256


def _dist_body(nx_ref, xt_ref, m_ref):
    src = nx_ref[0]
    dst = xt_ref[0]
    mm = lax.dot_general(src, dst, (((1,), (0,)), ((), ())),
                         preferred_element_type=jnp.float32)
    s0 = src[:, 0] * src[:, 0]
    s1 = src[:, 1] * src[:, 1]
    s2 = src[:, 2] * src[:, 2]
    a2 = ((s0 + s1) + s2).reshape(SB, 1)
    d0 = dst[0] * dst[0]
    d1 = dst[1] * dst[1]
    d2 = dst[2] * dst[2]
    b2 = ((d0 + d1) + d2).reshape(1, N)
    d = (-2.0 * mm + a2) + b2
    m_ref[...] = jnp.logical_not(d > RADIUS ** 2).astype(jnp.int32)


def _dist_mask(new_xyz, xyzB):
    nj = S // SB
    return pl.pallas_call(
        _dist_body,
        grid=(B, nj),
        in_specs=[
            pl.BlockSpec((1, SB, 3), lambda b, j: (b, j, 0)),
            pl.BlockSpec((1, 3, N), lambda b, j: (b, 0, 0)),
        ],
        out_specs=pl.BlockSpec((SB, N), lambda b, j: (b * nj + j, 0)),
        out_shape=jax.ShapeDtypeStruct((B * S, N), jnp.int32),
    )(new_xyz, xyzB)


ROWB = 8
NCH = CPW // ROWB


def _sc_ball(maskf):
    mesh = plsc.VectorSubcoreMesh(core_axis_name="c", subcore_axis_name="s")

    @functools.partial(
        pl.kernel, mesh=mesh, compiler_params=_sc_params(),
        out_type=jax.ShapeDtypeStruct((M,), jnp.int32),
        scratch_types=[
            pltpu.VMEM((ROWB * N,), jnp.int32),
            pltpu.VMEM((ROWB * N,), jnp.int32),
            pltpu.VMEM((192,), jnp.int32),
            pltpu.VMEM((CPW * K,), jnp.int32),
            pltpu.SemaphoreType.DMA,
            pltpu.SemaphoreType.DMA,
        ],
    )
    def k(maskf_hbm, idx_hbm, mb0, mb1, buf, obuf, s0, s1):
        wid = lax.axis_index("s") * 2 + lax.axis_index("c")
        b = wid // (NWORK // B)
        cb = wid * CPW
        iota = lax.iota(jnp.int32, 16)
        bN = b * N

        def chunk_src(ci):
            return maskf_hbm.at[pl.ds((cb + ci * ROWB) * N, ROWB * N)]

        UNROLL = 8

        def do_rows(mbuf, ci):
            @pl.loop(0, ROWB)
            def _(r):
                rbase = r * N

                def cond(st):
                    blk, cnt = st
                    return (cnt < K) & (blk < NCHUNK // UNROLL)

                def body(st):
                    blk, cnt = st
                    cntv = jnp.full((16,), cnt, jnp.int32)
                    base = rbase + blk * (UNROLL * 16)
                    for g in range(UNROLL):
                        sel = mbuf[pl.ds(base + g * 16, 16)]
                        selb = sel > 0
                        ps = plsc.cumsum(sel)
                        plsc.store_scatter(
                            buf, [(cntv - 1) + ps],
                            iota + (blk * (UNROLL * 16) + g * 16), mask=selb)
                        cntv = cntv + plsc.all_reduce_population_count(selb)
                    return blk + 1, jnp.max(cntv)

                _, cnt = lax.while_loop(cond, body,
                                        (jnp.int32(0), jnp.int32(0)))
                firstv = plsc.load_gather(buf, [jnp.zeros((16,), jnp.int32)])
                j = ci * ROWB + r
                for g in range(K // 16):
                    lanes = iota + (g * 16)
                    v = buf[pl.ds(g * 16, 16)]
                    outg = jnp.where(lanes < cnt, v, firstv) + bN
                    plsc.store_scatter(obuf, [lanes + (j * K)], outg)

        pltpu.async_copy(chunk_src(0), mb0, s0)

        @pl.loop(0, NCH, step=2)
        def _(ci):
            pltpu.async_copy(chunk_src(ci + 1), mb1, s1)
            pltpu.make_async_copy(chunk_src(ci), mb0, s0).wait()
            do_rows(mb0, ci)

            @pl.when(ci + 2 < NCH)
            def _():
                pltpu.async_copy(chunk_src(ci + 2), mb0, s0)

            pltpu.make_async_copy(chunk_src(ci + 1), mb1, s1).wait()
            do_rows(mb1, ci + 1)

        pltpu.sync_copy(obuf, idx_hbm.at[pl.ds(wid * CPW * K, CPW * K)])

    return k(maskf)


def _sc_gather(q, idx):
    mesh = plsc.VectorSubcoreMesh(core_axis_name="c", subcore_axis_name="s")
    RPW = M // NWORK
    CH = 128

    @functools.partial(
        pl.kernel, mesh=mesh, compiler_params=_sc_params(tc_tiling=False),
        out_type=jax.ShapeDtypeStruct((M, C1), jnp.float32),
        scratch_types=[
            pltpu.VMEM((RPW,), jnp.int32),
            pltpu.VMEM((CH, C1), jnp.float32),
            pltpu.VMEM((CH, C1), jnp.float32),
            pltpu.SemaphoreType.DMA,
            pltpu.SemaphoreType.DMA,
        ],
    )
    def k(q_hbm, idx_hbm, g_hbm, idxv, rb0, rb1, s0, s1):
        wid = lax.axis_index("s") * 2 + lax.axis_index("c")
        base = wid * RPW
        pltpu.sync_copy(idx_hbm.at[pl.ds(base, RPW)], idxv)

        @pl.loop(0, RPW // CH, step=2)
        def _(c):
            h0 = pltpu.async_copy(q_hbm.at[idxv.at[pl.ds(c * CH, CH)]], rb0, s0)
            h1 = pltpu.async_copy(
                q_hbm.at[idxv.at[pl.ds((c + 1) * CH, CH)]], rb1, s1)
            h0.wait()
            pltpu.sync_copy(rb0, g_hbm.at[pl.ds(base + c * CH, CH)])
            h1.wait()
            pltpu.sync_copy(rb1, g_hbm.at[pl.ds(base + (c + 1) * CH, CH)])

    return k(q, idx)


def kernel(xyz, points, W0, g0, b0, W1, g1, b1, W2, g2, b2):
    xyzT = jnp.transpose(xyz, (2, 0, 1))
    cx, cy, cz = _fps(xyzT)
    new_xyz = jnp.stack([cx.T, cy.T, cz.T], axis=-1)

    xyzB = jnp.transpose(xyz, (0, 2, 1))
    q = _q_premul(points, xyzB, W0)
    nxT = jnp.stack([cx.T.reshape(-1), cy.T.reshape(-1), cz.T.reshape(-1)])
    c0 = _c0_premul(nxT, W0)

    mask = _dist_mask(new_xyz, xyzB)
    flat_idx = _sc_ball(mask.reshape(-1))
    G = _sc_gather(q, flat_idx)
    pooled = _mlp(G, c0, g0, b0, W1, g1, b1, W2, g2, b2)
    out = jnp.transpose(pooled.reshape(B, S, C2), (0, 2, 1))
    return new_xyz, out
